# baseline (speedup 1.0000x reference)
import functools

import jax
import jax.numpy as jnp
from jax import lax
from jax.experimental import pallas as pl
from jax.experimental.pallas import tpu as pltpu
from jax.experimental.pallas import tpu_sc as plsc

UNITS = 128
B = 4096
N = 200
_NCHUNK = 2
_CB = B // _NCHUNK

_INFO = plsc.get_sparse_core_info()
_NC = _INFO.num_cores
_NS = _INFO.num_subcores
_NW = _NC * _NS
_BPW = _CB // _NW
_L = 16


def _sc_gather(table, first_ids, last_ids, chunk_start):
    mesh = plsc.VectorSubcoreMesh(core_axis_name="c", subcore_axis_name="s")

    @functools.partial(
        pl.kernel,
        mesh=mesh,
        out_type=(
            jax.ShapeDtypeStruct((_CB, UNITS), jnp.float32),
            jax.ShapeDtypeStruct((_CB, UNITS), jnp.float32),
        ),
        scratch_types=[
            pltpu.VMEM((_BPW,), jnp.int32),
            pltpu.VMEM((_BPW,), jnp.int32),
            pltpu.VMEM((_BPW, UNITS), jnp.float32),
            pltpu.VMEM((_BPW, UNITS), jnp.float32),
            pltpu.SemaphoreType.DMA,
            pltpu.SemaphoreType.DMA,
        ],
    )
    def k(table_hbm, gf_hbm, gl_hbm, outf_hbm, outl_hbm,
          idxf_v, idxl_v, rowsf_v, rowsl_v, semf, seml):
        wid = lax.axis_index("s") * _NC + lax.axis_index("c")
        base = wid * _BPW
        pltpu.sync_copy(gf_hbm.at[pl.ds(base, _BPW)], idxf_v)
        pltpu.sync_copy(gl_hbm.at[pl.ds(base, _BPW)], idxl_v)
        step = lax.iota(jnp.int32, _L) * N
        for j in range(_BPW // _L):
            off = (chunk_start + j * _L) * N + base * N
            sl = pl.ds(j * _L, _L)
            idxf_v[sl] = idxf_v[sl] + step + off
            idxl_v[sl] = idxl_v[sl] + step + off
        cpf = pltpu.async_copy(table_hbm.at[idxf_v], rowsf_v, semf)
        cpl = pltpu.async_copy(table_hbm.at[idxl_v], rowsl_v, seml)
        cpf.wait()
        cpl.wait()
        pltpu.sync_copy(rowsf_v, outf_hbm.at[pl.ds(base, _BPW)])
        pltpu.sync_copy(rowsl_v, outl_hbm.at[pl.ds(base, _BPW)])

    return k(table, first_ids, last_ids)


_BM = 512


def _proj_body(step_ref, f_ref, l_ref, fc_ref, wph_ref, w_ref, _, o_ref):
    use_ph = step_ref[0] == 1
    f = jnp.where(use_ph, jnp.broadcast_to(wph_ref[0:1, :], (_BM, UNITS)),
                  f_ref[...])
    l = jnp.where(use_ph, jnp.broadcast_to(wph_ref[1:2, :], (_BM, UNITS)),
                  l_ref[...])
    acc = jnp.dot(f, w_ref[:UNITS, :], preferred_element_type=jnp.float32)
    acc += jnp.dot(l, w_ref[UNITS:, :], preferred_element_type=jnp.float32)
    o_ref[...] = fc_ref[...] + acc


def _tc_project(step_arr, first_rows, last_rows, fixed, wph, w, out_buf,
                chunk):
    grid = (_CB // _BM,)
    blk0 = chunk * (_CB // _BM)
    in_row = pl.BlockSpec((_BM, UNITS), lambda i: (i, 0))
    out_row = pl.BlockSpec((_BM, UNITS), lambda i: (i + blk0, 0))
    return pl.pallas_call(
        _proj_body,
        grid=grid,
        in_specs=[
            pl.BlockSpec(memory_space=pltpu.SMEM),
            in_row, in_row, in_row,
            pl.BlockSpec((2, UNITS), lambda i: (0, 0)),
            pl.BlockSpec((2 * UNITS, UNITS), lambda i: (0, 0)),
            pl.BlockSpec(memory_space=pl.ANY),
        ],
        out_specs=out_row,
        out_shape=jax.ShapeDtypeStruct((B, UNITS), jnp.float32),
        input_output_aliases={6: 0},
    )(step_arr, first_rows, last_rows, fixed, wph, w, out_buf)


def kernel(node_embed, fixed_context, first_node, last_node, step_count,
           W_context_placeholder, W_dense):
    table = node_embed.reshape(B * N, UNITS)
    first_ids = first_node.reshape(B).astype(jnp.int32)
    last_ids = last_node.reshape(B).astype(jnp.int32)

    step_arr = jnp.asarray(step_count, jnp.int32).reshape(1)
    wph = W_context_placeholder.reshape(2, UNITS)
    fixed = fixed_context.reshape(B, UNITS)

    gathered = [
        _sc_gather(table,
                   lax.slice(first_ids, (c * _CB,), ((c + 1) * _CB,)),
                   lax.slice(last_ids, (c * _CB,), ((c + 1) * _CB,)),
                   c * _CB)
        for c in range(_NCHUNK)
    ]

    out = jnp.empty((B, UNITS), jnp.float32)
    for c in range(_NCHUNK):
        f_rows, l_rows = gathered[c]
        fc = lax.slice(fixed, (c * _CB, 0), ((c + 1) * _CB, UNITS))
        out = _tc_project(step_arr, f_rows, l_rows, fc, wph, W_dense, out, c)
    return out.reshape(B, 1, UNITS)


# speedup vs baseline: 1.2032x; 1.2032x over previous
import functools

import jax
import jax.numpy as jnp
from jax import lax
from jax.experimental import pallas as pl
from jax.experimental.pallas import tpu as pltpu
from jax.experimental.pallas import tpu_sc as plsc

UNITS = 128
B = 4096
N = 200

_INFO = plsc.get_sparse_core_info()
_NC = _INFO.num_cores
_NS = _INFO.num_subcores
_NW = _NC * _NS
_BPW = B // _NW
_L = 16


def _sc_gather(table, first_ids, last_ids):
    mesh = plsc.VectorSubcoreMesh(core_axis_name="c", subcore_axis_name="s")

    @functools.partial(
        pl.kernel,
        mesh=mesh,
        out_type=(
            jax.ShapeDtypeStruct((B, UNITS), jnp.float32),
            jax.ShapeDtypeStruct((B, UNITS), jnp.float32),
        ),
        scratch_types=[
            pltpu.VMEM((_BPW,), jnp.int32),
            pltpu.VMEM((_BPW,), jnp.int32),
            pltpu.VMEM((_BPW, UNITS), jnp.float32),
            pltpu.VMEM((_BPW, UNITS), jnp.float32),
            pltpu.SemaphoreType.DMA,
            pltpu.SemaphoreType.DMA,
            pltpu.SemaphoreType.DMA,
            pltpu.SemaphoreType.DMA,
        ],
    )
    def k(table_hbm, gf_hbm, gl_hbm, outf_hbm, outl_hbm,
          idxf_v, idxl_v, rowsf_v, rowsl_v, semf, seml, semof, semol):
        wid = lax.axis_index("s") * _NC + lax.axis_index("c")
        base = wid * _BPW
        pltpu.sync_copy(gf_hbm.at[pl.ds(base, _BPW)], idxf_v)
        pltpu.sync_copy(gl_hbm.at[pl.ds(base, _BPW)], idxl_v)
        step = lax.iota(jnp.int32, _L) * N
        for j in range(_BPW // _L):
            off = (base + j * _L) * N
            sl = pl.ds(j * _L, _L)
            idxf_v[sl] = idxf_v[sl] + step + off
            idxl_v[sl] = idxl_v[sl] + step + off
        cpf = pltpu.async_copy(table_hbm.at[idxf_v], rowsf_v, semf)
        cpl = pltpu.async_copy(table_hbm.at[idxl_v], rowsl_v, seml)
        cpf.wait()
        cof = pltpu.async_copy(rowsf_v, outf_hbm.at[pl.ds(base, _BPW)], semof)
        cpl.wait()
        col = pltpu.async_copy(rowsl_v, outl_hbm.at[pl.ds(base, _BPW)], semol)
        cof.wait()
        col.wait()

    return k(table, first_ids, last_ids)


_BM = 1024


def _proj_body(step_ref, f_ref, l_ref, fc_ref, wph_ref, w_ref, o_ref):
    use_ph = step_ref[0] == 1
    f = jnp.where(use_ph, jnp.broadcast_to(wph_ref[0:1, :], (_BM, UNITS)),
                  f_ref[...])
    l = jnp.where(use_ph, jnp.broadcast_to(wph_ref[1:2, :], (_BM, UNITS)),
                  l_ref[...])
    acc = jnp.dot(f, w_ref[:UNITS, :], preferred_element_type=jnp.float32)
    acc += jnp.dot(l, w_ref[UNITS:, :], preferred_element_type=jnp.float32)
    o_ref[...] = fc_ref[...] + acc


def _tc_project(step_arr, first_rows, last_rows, fixed, wph, w):
    grid = (B // _BM,)
    row_spec = pl.BlockSpec((_BM, UNITS), lambda i: (i, 0))
    return pl.pallas_call(
        _proj_body,
        grid=grid,
        in_specs=[
            pl.BlockSpec(memory_space=pltpu.SMEM),
            row_spec, row_spec, row_spec,
            pl.BlockSpec((2, UNITS), lambda i: (0, 0)),
            pl.BlockSpec((2 * UNITS, UNITS), lambda i: (0, 0)),
        ],
        out_specs=row_spec,
        out_shape=jax.ShapeDtypeStruct((B, UNITS), jnp.float32),
    )(step_arr, first_rows, last_rows, fixed, wph, w)


def kernel(node_embed, fixed_context, first_node, last_node, step_count,
           W_context_placeholder, W_dense):
    table = node_embed.reshape(B * N, UNITS)
    first_ids = first_node.reshape(B).astype(jnp.int32)
    last_ids = last_node.reshape(B).astype(jnp.int32)

    first_rows, last_rows = _sc_gather(table, first_ids, last_ids)

    step_arr = jnp.asarray(step_count, jnp.int32).reshape(1)
    wph = W_context_placeholder.reshape(2, UNITS)
    fixed = fixed_context.reshape(B, UNITS)

    out = _tc_project(step_arr, first_rows, last_rows, fixed, wph, W_dense)
    return out.reshape(B, 1, UNITS)
